# Initial kernel scaffold; baseline (speedup 1.0000x reference)
import jax
import jax.numpy as jnp
from jax.experimental import pallas as pl


def kernel(x, edge_index, edge_attr, W, b, g1, b1, g2, b2):
    raise NotImplementedError("write your pallas kernel here")


# speedup vs baseline: 2.6112x; 2.6112x over previous
import functools

import jax
import jax.numpy as jnp
from jax import lax
from jax.experimental import pallas as pl
from jax.experimental.pallas import tpu as pltpu
from jax.experimental.pallas import tpu_sc as plsc

N = 10000
NUM_NBR = 32
ATOM = 128
NBR_F = 16
E = N * NUM_NBR
F2 = 2 * ATOM
EPS = 1e-5

NW = 32
PER_W = E // NW
CH = 80
N_CH = PER_W // CH

G = 80
EB = G * NUM_NBR
NBLK = N // G


def _sigmoid(v):
    return 1.0 / (1.0 + jnp.exp(-v))


def _softplus(v):
    return jnp.maximum(v, 0.0) + jnp.log1p(jnp.exp(-jnp.abs(v)))


def _sc_gather(x, dst):
    mesh = plsc.VectorSubcoreMesh(core_axis_name="c", subcore_axis_name="s")

    @functools.partial(
        pl.kernel,
        mesh=mesh,
        out_type=jax.ShapeDtypeStruct((E, ATOM), x.dtype),
        scratch_types=[
            pltpu.VMEM((CH,), jnp.int32),
            pltpu.VMEM((CH, ATOM), x.dtype),
            pltpu.SemaphoreType.DMA,
        ],
    )
    def gk(x_hbm, dst_hbm, out_hbm, idx_v, rows_v, sem):
        wid = lax.axis_index("s") * 2 + lax.axis_index("c")
        base = wid * PER_W

        def body(i, carry):
            off = base + i * CH
            pltpu.sync_copy(dst_hbm.at[pl.ds(off, CH)], idx_v)
            pltpu.async_copy(x_hbm.at[idx_v], rows_v, sem).wait()
            pltpu.sync_copy(rows_v, out_hbm.at[pl.ds(off, CH)])
            return carry

        lax.fori_loop(0, N_CH, body, 0)

    return gk(x, dst)


def _k_stats(xg, ea, x, wd, we, ws, bb):
    def body(xg_r, ea_r, x_r, wd_r, we_r, ws_r, bb_r, s_ref, q_ref):
        m = jnp.dot(xg_r[...], wd_r[...], preferred_element_type=jnp.float32)
        m = m + jnp.dot(ea_r[...], we_r[...], preferred_element_type=jnp.float32)
        a = jnp.dot(x_r[...], ws_r[...], preferred_element_type=jnp.float32)
        h = m.reshape(G, NUM_NBR, F2) + a[:, None, :] + bb_r[...][None]

        @pl.when(pl.program_id(0) == 0)
        def _():
            s_ref[...] = jnp.zeros_like(s_ref)
            q_ref[...] = jnp.zeros_like(q_ref)

        s_ref[...] += jnp.sum(h, axis=(0, 1))[None, :]
        q_ref[...] += jnp.sum(h * h, axis=(0, 1))[None, :]

    return pl.pallas_call(
        body,
        grid=(NBLK,),
        in_specs=[
            pl.BlockSpec((EB, ATOM), lambda i: (i, 0)),
            pl.BlockSpec((EB, NBR_F), lambda i: (i, 0)),
            pl.BlockSpec((G, ATOM), lambda i: (i, 0)),
            pl.BlockSpec((ATOM, F2), lambda i: (0, 0)),
            pl.BlockSpec((NBR_F, F2), lambda i: (0, 0)),
            pl.BlockSpec((ATOM, F2), lambda i: (0, 0)),
            pl.BlockSpec((1, F2), lambda i: (0, 0)),
        ],
        out_specs=[
            pl.BlockSpec((1, F2), lambda i: (0, 0)),
            pl.BlockSpec((1, F2), lambda i: (0, 0)),
        ],
        out_shape=[
            jax.ShapeDtypeStruct((1, F2), jnp.float32),
            jax.ShapeDtypeStruct((1, F2), jnp.float32),
        ],
    )(xg, ea, x, wd, we, ws, bb)


def _k_main(xg, ea, x, wd, we, ws, bb, s, q, g1, b1):
    def body(xg_r, ea_r, x_r, wd_r, we_r, ws_r, bb_r, s_r, q_r, g1_r, b1_r,
             out_ref, s2_ref, q2_ref):
        mu = s_r[...] * (1.0 / E)
        var = q_r[...] * (1.0 / E) - mu * mu
        inv = lax.rsqrt(var + EPS)
        scale = g1_r[...] * inv
        shift = b1_r[...] - mu * scale

        m = jnp.dot(xg_r[...], wd_r[...], preferred_element_type=jnp.float32)
        m = m + jnp.dot(ea_r[...], we_r[...], preferred_element_type=jnp.float32)
        a = jnp.dot(x_r[...], ws_r[...], preferred_element_type=jnp.float32)
        h = m.reshape(G, NUM_NBR, F2) + a[:, None, :] + bb_r[...][None]
        hn = h * scale[None] + shift[None]
        filt = _sigmoid(hn[:, :, :ATOM])
        core = _softplus(hn[:, :, ATOM:])
        ssum = jnp.sum(filt * core, axis=1)
        out_ref[...] = ssum

        @pl.when(pl.program_id(0) == 0)
        def _():
            s2_ref[...] = jnp.zeros_like(s2_ref)
            q2_ref[...] = jnp.zeros_like(q2_ref)

        s2_ref[...] += jnp.sum(ssum, axis=0)[None, :]
        q2_ref[...] += jnp.sum(ssum * ssum, axis=0)[None, :]

    return pl.pallas_call(
        body,
        grid=(NBLK,),
        in_specs=[
            pl.BlockSpec((EB, ATOM), lambda i: (i, 0)),
            pl.BlockSpec((EB, NBR_F), lambda i: (i, 0)),
            pl.BlockSpec((G, ATOM), lambda i: (i, 0)),
            pl.BlockSpec((ATOM, F2), lambda i: (0, 0)),
            pl.BlockSpec((NBR_F, F2), lambda i: (0, 0)),
            pl.BlockSpec((ATOM, F2), lambda i: (0, 0)),
            pl.BlockSpec((1, F2), lambda i: (0, 0)),
            pl.BlockSpec((1, F2), lambda i: (0, 0)),
            pl.BlockSpec((1, F2), lambda i: (0, 0)),
            pl.BlockSpec((1, F2), lambda i: (0, 0)),
            pl.BlockSpec((1, F2), lambda i: (0, 0)),
        ],
        out_specs=[
            pl.BlockSpec((G, ATOM), lambda i: (i, 0)),
            pl.BlockSpec((1, ATOM), lambda i: (0, 0)),
            pl.BlockSpec((1, ATOM), lambda i: (0, 0)),
        ],
        out_shape=[
            jax.ShapeDtypeStruct((N, ATOM), jnp.float32),
            jax.ShapeDtypeStruct((1, ATOM), jnp.float32),
            jax.ShapeDtypeStruct((1, ATOM), jnp.float32),
        ],
    )(xg, ea, x, wd, we, ws, bb, s, q, g1, b1)


def _k_finish(x, ssum, s2, q2, g2, b2):
    def body(x_r, ss_r, s2_r, q2_r, g2_r, b2_r, out_ref):
        mu2 = s2_r[...] * (1.0 / N)
        var2 = q2_r[...] * (1.0 / N) - mu2 * mu2
        inv2 = lax.rsqrt(var2 + EPS)
        bn2 = (ss_r[...] - mu2) * (inv2 * g2_r[...]) + b2_r[...]
        out_ref[...] = _softplus(x_r[...] + bn2)

    return pl.pallas_call(
        body,
        grid=(NBLK,),
        in_specs=[
            pl.BlockSpec((G, ATOM), lambda i: (i, 0)),
            pl.BlockSpec((G, ATOM), lambda i: (i, 0)),
            pl.BlockSpec((1, ATOM), lambda i: (0, 0)),
            pl.BlockSpec((1, ATOM), lambda i: (0, 0)),
            pl.BlockSpec((1, ATOM), lambda i: (0, 0)),
            pl.BlockSpec((1, ATOM), lambda i: (0, 0)),
        ],
        out_specs=pl.BlockSpec((G, ATOM), lambda i: (i, 0)),
        out_shape=jax.ShapeDtypeStruct((N, ATOM), jnp.float32),
    )(x, ssum, s2, q2, g2, b2)


def kernel(x, edge_index, edge_attr, W, b, g1, b1, g2, b2):
    dst = edge_index[1].astype(jnp.int32)
    Wt = W.T
    wd = Wt[:ATOM]
    ws = Wt[ATOM:2 * ATOM]
    we = Wt[2 * ATOM:]
    bb = b.reshape(1, F2)
    g1r = g1.reshape(1, F2)
    b1r = b1.reshape(1, F2)
    g2r = g2.reshape(1, ATOM)
    b2r = b2.reshape(1, ATOM)

    xg = _sc_gather(x, dst)
    s, q = _k_stats(xg, edge_attr, x, wd, we, ws, bb)
    ssum, s2, q2 = _k_main(xg, edge_attr, x, wd, we, ws, bb, s, q, g1r, b1r)
    return _k_finish(x, ssum, s2, q2, g2r, b2r)
